# baseline (speedup 1.0000x reference)
import jax
import jax.numpy as jnp
from jax import lax
from jax.experimental import pallas as pl
from jax.experimental.pallas import tpu as pltpu
from jax.experimental.pallas import tpu_sc as plsc
import functools

_INDEX = 1024


def _make_sc_select(b, d, dtype):
    mesh = plsc.VectorSubcoreMesh(core_axis_name="c", subcore_axis_name="s")

    @functools.partial(
        pl.kernel,
        mesh=mesh,
        out_type=jax.ShapeDtypeStruct((b, d), dtype),
        scratch_types=[
            pltpu.VMEM((d,), dtype),
        ],
    )
    def sc_select(x_hbm, out_hbm, row_v):
        num_cores = lax.axis_size("c")
        wid = lax.axis_index("s") * num_cores + lax.axis_index("c")

        @pl.when(wid < b)
        def _():
            pltpu.sync_copy(x_hbm.at[wid, _INDEX], row_v)
            pltpu.sync_copy(row_v, out_hbm.at[wid])

    return sc_select


def kernel(x):
    b, s, d = x.shape
    return _make_sc_select(b, d, x.dtype)(x)
